# Initial kernel scaffold; baseline (speedup 1.0000x reference)
import functools

import jax
import jax.numpy as jnp
from jax import lax
from jax.experimental import pallas as pl
from jax.experimental.pallas import tpu as pltpu
from jax.experimental.pallas import tpu_sc as plsc

_NC = 2
_NS = 16
_NW = _NC * _NS
_LANES = 16
_CHUNK = 100


@functools.lru_cache(maxsize=None)
def _build(batch, seqlen, vocab, embed):
    rows = batch * seqlen
    assert rows % (_NW * _CHUNK) == 0
    rpw = rows // _NW
    nsub = rpw // _CHUNK
    assert rpw % seqlen == 0
    assert seqlen % _CHUNK == 0
    assert embed % _LANES == 0
    nq = embed // _LANES
    noff = seqlen // _CHUNK

    mesh = plsc.VectorSubcoreMesh(core_axis_name="c", subcore_axis_name="s")

    @functools.partial(
        pl.kernel,
        out_type=jax.ShapeDtypeStruct((rows, embed), jnp.float32),
        mesh=mesh,
        scratch_types=[
            pltpu.VMEM((nsub, _CHUNK), jnp.int32),
            pltpu.VMEM((seqlen, embed), jnp.float32),
            pltpu.VMEM((_CHUNK, embed), jnp.float32),
            pltpu.SemaphoreType.DMA,
        ],
    )
    def fused(x_hbm, tok_hbm, pos_hbm, out_hbm, idx_v, pos_v, rows_v, gsem):
        cid = lax.axis_index("c")
        sid = lax.axis_index("s")
        wid = sid * _NC + cid
        pltpu.sync_copy(x_hbm.at[wid], idx_v)
        pltpu.sync_copy(pos_hbm, pos_v)

        def subchunk(r, carry):
            pltpu.async_copy(tok_hbm.at[idx_v.at[r]], rows_v, gsem).wait()
            off = lax.rem(r, noff) * _CHUNK

            def addrow(j, c2):
                for q in range(nq):
                    sl = pl.ds(q * _LANES, _LANES)
                    rows_v[j, sl] = rows_v[j, sl] + pos_v[off + j, sl]
                return c2

            lax.fori_loop(0, _CHUNK, addrow, None)
            pltpu.sync_copy(
                rows_v, out_hbm.at[pl.ds(wid * rpw + r * _CHUNK, _CHUNK)]
            )
            return carry

        lax.fori_loop(0, nsub, subchunk, None)

    return fused


def kernel(x, token_table, pos_table):
    batch, seqlen = x.shape
    vocab, embed = token_table.shape
    fused = _build(batch, seqlen, vocab, embed)
    rows = batch * seqlen
    x3 = x.astype(jnp.int32).reshape(_NW, rows // (_NW * _CHUNK), _CHUNK)
    out = fused(x3, token_table, pos_table)
    return out.reshape(batch, seqlen, embed)


# speedup vs baseline: 1.8964x; 1.8964x over previous
import functools

import jax
import jax.numpy as jnp
from jax import lax
from jax.experimental import pallas as pl
from jax.experimental.pallas import tpu as pltpu
from jax.experimental.pallas import tpu_sc as plsc

_NC = 2
_NS = 16
_NW = _NC * _NS
_LANES = 16
_CHUNK = 100


@functools.lru_cache(maxsize=None)
def _build(batch, seqlen, vocab, embed):
    rows = batch * seqlen
    assert rows % (_NW * _CHUNK) == 0
    rpw = rows // _NW
    nsub = rpw // _CHUNK
    assert rpw % seqlen == 0
    assert seqlen % _CHUNK == 0
    assert embed % _LANES == 0
    nq = embed // _LANES
    noff = seqlen // _CHUNK

    mesh = plsc.VectorSubcoreMesh(core_axis_name="c", subcore_axis_name="s")

    @functools.partial(
        pl.kernel,
        out_type=jax.ShapeDtypeStruct((rows // _CHUNK, _CHUNK, embed), jnp.float32),
        mesh=mesh,
        compiler_params=pltpu.CompilerParams(use_tc_tiling_on_sc=False),
        scratch_types=[
            pltpu.VMEM((nsub, _CHUNK), jnp.int32),
            pltpu.VMEM((seqlen, embed), jnp.float32),
            pltpu.VMEM((_CHUNK, embed), jnp.float32),
            pltpu.SemaphoreType.DMA,
        ],
    )
    def fused(x_hbm, tok_hbm, pos_hbm, out_hbm, idx_v, pos_v, rows_v, gsem):
        cid = lax.axis_index("c")
        sid = lax.axis_index("s")
        wid = sid * _NC + cid
        pltpu.sync_copy(x_hbm.at[wid], idx_v)
        pltpu.sync_copy(pos_hbm, pos_v)

        def subchunk(r, carry):
            pltpu.async_copy(tok_hbm.at[idx_v.at[r]], rows_v, gsem).wait()
            off = lax.rem(r, noff) * _CHUNK

            def addrow(j, c2):
                for q in range(nq):
                    sl = pl.ds(q * _LANES, _LANES)
                    rows_v[j, sl] = rows_v[j, sl] + pos_v[off + j, sl]
                return c2

            lax.fori_loop(0, _CHUNK, addrow, None)
            pltpu.sync_copy(rows_v, out_hbm.at[wid * nsub + r])
            return carry

        lax.fori_loop(0, nsub, subchunk, None)

    return fused


def kernel(x, token_table, pos_table):
    batch, seqlen = x.shape
    vocab, embed = token_table.shape
    fused = _build(batch, seqlen, vocab, embed)
    rows = batch * seqlen
    x3 = x.astype(jnp.int32).reshape(_NW, rows // (_NW * _CHUNK), _CHUNK)
    out = fused(x3, token_table, pos_table)
    return out.reshape(batch, seqlen, embed)


# speedup vs baseline: 3.0776x; 1.6229x over previous
import functools

import jax
import jax.numpy as jnp
from jax import lax
from jax.experimental import pallas as pl
from jax.experimental.pallas import tpu as pltpu
from jax.experimental.pallas import tpu_sc as plsc

_NC = 2
_NS = 16
_NW = _NC * _NS
_LANES = 16
_CHUNK = 100
_NBUF = 4
_PRE = 2


@functools.lru_cache(maxsize=None)
def _build(batch, seqlen, vocab, embed):
    rows = batch * seqlen
    assert rows % (_NW * _CHUNK) == 0
    rpw = rows // _NW
    nsub = rpw // _CHUNK
    assert nsub % _NBUF == 0
    assert rpw % seqlen == 0
    assert seqlen % _CHUNK == 0
    assert embed % _LANES == 0
    nq = embed // _LANES
    noff = seqlen // _CHUNK

    mesh = plsc.VectorSubcoreMesh(core_axis_name="c", subcore_axis_name="s")

    @functools.partial(
        pl.kernel,
        out_type=jax.ShapeDtypeStruct((rows // _CHUNK, _CHUNK, embed), jnp.float32),
        mesh=mesh,
        compiler_params=pltpu.CompilerParams(use_tc_tiling_on_sc=False),
        scratch_types=[
            pltpu.VMEM((nsub, _CHUNK), jnp.int32),
            pltpu.VMEM((seqlen, embed), jnp.float32),
            pltpu.VMEM((_NBUF, _CHUNK, embed), jnp.float32),
            pltpu.SemaphoreType.DMA((_NBUF,)),
            pltpu.SemaphoreType.DMA((_NBUF,)),
        ],
    )
    def fused(x_hbm, tok_hbm, pos_hbm, out_hbm, idx_v, pos_v, rows_v, gsem, osem):
        cid = lax.axis_index("c")
        sid = lax.axis_index("s")
        wid = sid * _NC + cid
        pltpu.sync_copy(x_hbm.at[wid], idx_v)
        pltpu.sync_copy(pos_hbm, pos_v)

        def gstart(t, b):
            pltpu.async_copy(tok_hbm.at[idx_v.at[t]], rows_v.at[b], gsem.at[b])

        def gwait(t, b):
            pltpu.make_async_copy(
                tok_hbm.at[idx_v.at[t]], rows_v.at[b], gsem.at[b]
            ).wait()

        def ostart(t, b):
            pltpu.async_copy(rows_v.at[b], out_hbm.at[wid * nsub + t], osem.at[b])

        def owait(b):
            pltpu.make_async_copy(rows_v.at[b], out_hbm.at[0], osem.at[b]).wait()

        for b in range(_PRE):
            gstart(b, b)

        def outer(i, carry):
            t0 = i * _NBUF
            for b in range(_NBUF):
                t = t0 + b
                gwait(t, b)
                off = lax.rem(t, noff) * _CHUNK

                def addrow(j, c2):
                    for q in range(nq):
                        sl = pl.ds(q * _LANES, _LANES)
                        rows_v[b, j, sl] = rows_v[b, j, sl] + pos_v[off + j, sl]
                    return c2

                lax.fori_loop(0, _CHUNK, addrow, None)

                ostart(t, b)
                u = t + _PRE
                bu = (b + _PRE) % _NBUF

                @pl.when(u < nsub)
                def _():
                    @pl.when(u >= _NBUF)
                    def _():
                        owait(bu)

                    gstart(u, bu)

            return carry

        lax.fori_loop(0, nsub // _NBUF, outer, None)
        for b in range(_NBUF):
            owait(b)

    return fused


def kernel(x, token_table, pos_table):
    batch, seqlen = x.shape
    vocab, embed = token_table.shape
    fused = _build(batch, seqlen, vocab, embed)
    rows = batch * seqlen
    x3 = x.astype(jnp.int32).reshape(_NW, rows // (_NW * _CHUNK), _CHUNK)
    out = fused(x3, token_table, pos_table)
    return out.reshape(batch, seqlen, embed)
